# Initial kernel scaffold; baseline (speedup 1.0000x reference)
import jax
import jax.numpy as jnp
from jax.experimental import pallas as pl


def kernel(x_cell, x_drug, edge_index_c2d, edge_weight_c2d, edge_index_d2c, edge_weight_d2c, edge_label_index, W_rel1_cd, W_root1_cd, b1_cd, W_rel1_dc, W_root1_dc, b1_dc, W_rel2_cd, W_root2_cd, b2_cd, W_rel2_dc, W_root2_dc, b2_dc):
    raise NotImplementedError("write your pallas kernel here")


# speedup vs baseline: 1.0044x; 1.0044x over previous
import functools

import jax
import jax.numpy as jnp
from jax.experimental import pallas as pl
from jax.experimental.pallas import tpu as pltpu

N_CELL = 10000
N_DRUG = 10000
E = 160000
D_IN = 256
H = 512


def _mm_body(relu, a_ref, b_ref, w1_ref, w2_ref, bias_ref, o_ref):
    acc = jnp.dot(a_ref[...], w1_ref[...], preferred_element_type=jnp.float32)
    acc += jnp.dot(b_ref[...], w2_ref[...], preferred_element_type=jnp.float32)
    acc += bias_ref[...]
    o_ref[...] = jnp.maximum(acc, 0.0) if relu else acc


def _fused_linear(agg, x, w_rel, w_root, bias, relu):
    n, k = agg.shape
    h = w_rel.shape[1]
    blk = 1000
    grid = (n // blk,)
    return pl.pallas_call(
        functools.partial(_mm_body, relu),
        grid=grid,
        in_specs=[
            pl.BlockSpec((blk, k), lambda i: (i, 0)),
            pl.BlockSpec((blk, k), lambda i: (i, 0)),
            pl.BlockSpec((k, h), lambda i: (0, 0)),
            pl.BlockSpec((k, h), lambda i: (0, 0)),
            pl.BlockSpec((1, h), lambda i: (0, 0)),
        ],
        out_specs=pl.BlockSpec((blk, h), lambda i: (i, 0)),
        out_shape=jax.ShapeDtypeStruct((n, h), jnp.float32),
    )(agg, x, w_rel, w_root, bias.reshape(1, h))


def _seg_sum(x_src, src, dst, w, num_dst):
    msg = jnp.take(x_src, src, axis=0) * w[:, None]
    return jax.ops.segment_sum(msg, dst, num_segments=num_dst)


def kernel(x_cell, x_drug, edge_index_c2d, edge_weight_c2d, edge_index_d2c,
           edge_weight_d2c, edge_label_index, W_rel1_cd, W_root1_cd, b1_cd,
           W_rel1_dc, W_root1_dc, b1_dc, W_rel2_cd, W_root2_cd, b2_cd,
           W_rel2_dc, W_root2_dc, b2_dc):
    src_cd, dst_cd = edge_index_c2d[0], edge_index_c2d[1]
    src_dc, dst_dc = edge_index_d2c[0], edge_index_d2c[1]

    agg_d1 = _seg_sum(x_cell, src_cd, dst_cd, edge_weight_c2d, N_DRUG)
    agg_c1 = _seg_sum(x_drug, src_dc, dst_dc, edge_weight_d2c, N_CELL)
    z_drug1 = _fused_linear(agg_d1, x_drug, W_rel1_cd, W_root1_cd, b1_cd, True)
    z_cell1 = _fused_linear(agg_c1, x_cell, W_rel1_dc, W_root1_dc, b1_dc, True)

    agg_d2 = _seg_sum(z_cell1, src_cd, dst_cd, edge_weight_c2d, N_DRUG)
    agg_c2 = _seg_sum(z_drug1, src_dc, dst_dc, edge_weight_d2c, N_CELL)
    z_drug2 = _fused_linear(agg_d2, z_drug1, W_rel2_cd, W_root2_cd, b2_cd, False)
    z_cell2 = _fused_linear(agg_c2, z_cell1, W_rel2_dc, W_root2_dc, b2_dc, False)

    row1 = edge_label_index[0]
    row2 = edge_label_index[1]
    return (jnp.take(z_cell2, row1, axis=0) * jnp.take(z_drug2, row2, axis=0)).sum(axis=-1)


# speedup vs baseline: 1.7015x; 1.6940x over previous
import functools

import jax
import jax.numpy as jnp
from jax import lax
from jax.experimental import pallas as pl
from jax.experimental.pallas import tpu as pltpu
from jax.experimental.pallas import tpu_sc as plsc

N_CELL = 10000
N_DRUG = 10000
E = 160000
D_IN = 256
H = 512

NC = 2
NS = 16
LANES = 16
NP = 10240


def _mm_body(relu, ng, a_ref, b_ref, w1_ref, w2_ref, bias_ref, o_ref):
    acc = jnp.dot(a_ref[0], w1_ref[0], preferred_element_type=jnp.float32)
    for g in range(1, ng):
        acc += jnp.dot(a_ref[g], w1_ref[g], preferred_element_type=jnp.float32)
    acc += jnp.dot(b_ref[...], w2_ref[...], preferred_element_type=jnp.float32)
    acc += bias_ref[...]
    o_ref[...] = jnp.maximum(acc, 0.0) if relu else acc


def _fused_linear(agg3, x, w_rel, w_root, bias, relu):
    ng, n, _ = agg3.shape
    k = ng * 128
    h = w_rel.shape[1]
    w3 = w_rel.reshape(ng, 128, h)
    blk = 1024
    grid = (n // blk,)
    return pl.pallas_call(
        functools.partial(_mm_body, relu, ng),
        grid=grid,
        in_specs=[
            pl.BlockSpec((ng, blk, 128), lambda i: (0, i, 0)),
            pl.BlockSpec((blk, k), lambda i: (i, 0)),
            pl.BlockSpec((ng, 128, h), lambda i: (0, 0, 0)),
            pl.BlockSpec((k, h), lambda i: (0, 0)),
            pl.BlockSpec((1, h), lambda i: (0, 0)),
        ],
        out_specs=pl.BlockSpec((blk, h), lambda i: (i, 0)),
        out_shape=jax.ShapeDtypeStruct((n, h), jnp.float32),
    )(agg3, x, w3, w_root, bias.reshape(1, h))


def _sc_segsum_call(num_dst, G):
    P = G // NC
    EP = E // NS
    K = 80
    NCH = EP // K
    RPT = num_dst // NS
    mesh = plsc.VectorSubcoreMesh(core_axis_name="c", subcore_axis_name="s")

    @functools.partial(
        pl.kernel,
        out_type=jax.ShapeDtypeStruct((G, num_dst, 128), jnp.float32),
        mesh=mesh,
        scratch_types=[
            pltpu.VMEM((K,), jnp.int32),
            pltpu.VMEM((K,), jnp.int32),
            pltpu.VMEM((K,), jnp.int32),
            pltpu.VMEM((K,), jnp.float32),
            pltpu.VMEM((K, 128), jnp.float32),
            pltpu.VMEM_SHARED((num_dst, 128), jnp.float32),
            pltpu.SemaphoreType.DMA,
        ],
    )
    def k(xv_h, src_h, dst_h, w_h, out_h, sidx, didx, gidx, wbuf, rows,
          acc, sem):
        c = lax.axis_index("c")
        s = lax.axis_index("s")
        tbase = s * EP
        rbase = s * RPT

        for p in range(P):
            g = p * NC + c

            def zb(i, _):
                for j in range(8):
                    rows[i, pl.ds(16 * j, 16)] = jnp.zeros((16,), jnp.float32)
                return _
            lax.fori_loop(0, K, zb, None)
            nz_full = RPT // K
            for q in range(nz_full):
                pltpu.sync_copy(rows, acc.at[pl.ds(rbase + q * K, K)])
            rem = RPT - nz_full * K
            if rem:
                pltpu.sync_copy(rows.at[pl.ds(0, rem)],
                                acc.at[pl.ds(rbase + nz_full * K, rem)])
            plsc.subcore_barrier()

            def chunk(ch, _):
                base = tbase + ch * K
                pltpu.sync_copy(src_h.at[pl.ds(base, K)], sidx)
                pltpu.sync_copy(dst_h.at[pl.ds(base, K)], didx)
                pltpu.sync_copy(w_h.at[pl.ds(base, K)], wbuf)

                def gix(i, _):
                    v = sidx[pl.ds(i * 16, 16)]
                    gidx[pl.ds(i * 16, 16)] = v * G + g
                    return _
                lax.fori_loop(0, K // 16, gix, None)
                pltpu.async_copy(xv_h.at[gidx], rows, sem).wait()

                def scale(i, _):
                    w16 = wbuf[pl.ds(i * 16, 16)]
                    for l in range(16):
                        wl = w16[l]
                        e = i * 16 + l
                        for j in range(8):
                            rows[e, pl.ds(16 * j, 16)] = (
                                rows[e, pl.ds(16 * j, 16)] * wl)
                    return _
                lax.fori_loop(0, K // 16, scale, None)
                pltpu.sync_copy(rows, acc.at[didx], add=True)
                return _
            lax.fori_loop(0, NCH, chunk, None)
            plsc.subcore_barrier()
            pltpu.sync_copy(acc.at[pl.ds(rbase, RPT)],
                            out_h.at[g, pl.ds(rbase, RPT)])
            if p + 1 < P:
                plsc.subcore_barrier()

    return k


def _seg_sum3(x_src, src, dst, w, num_dst):
    n_src, d = x_src.shape
    G = d // 128
    xv = x_src.reshape(n_src * G, 128)
    return _sc_segsum_call(num_dst, G)(xv, src, dst, w)


def kernel(x_cell, x_drug, edge_index_c2d, edge_weight_c2d, edge_index_d2c,
           edge_weight_d2c, edge_label_index, W_rel1_cd, W_root1_cd, b1_cd,
           W_rel1_dc, W_root1_dc, b1_dc, W_rel2_cd, W_root2_cd, b2_cd,
           W_rel2_dc, W_root2_dc, b2_dc):
    src_cd, dst_cd = edge_index_c2d[0], edge_index_c2d[1]
    src_dc, dst_dc = edge_index_d2c[0], edge_index_d2c[1]

    x_cell = jnp.pad(x_cell, ((0, NP - N_CELL), (0, 0)))
    x_drug = jnp.pad(x_drug, ((0, NP - N_DRUG), (0, 0)))

    agg_d1 = _seg_sum3(x_cell, src_cd, dst_cd, edge_weight_c2d, NP)
    agg_c1 = _seg_sum3(x_drug, src_dc, dst_dc, edge_weight_d2c, NP)
    agg_d1, agg_c1 = lax.optimization_barrier((agg_d1, agg_c1))
    z_drug1 = _fused_linear(agg_d1, x_drug, W_rel1_cd, W_root1_cd, b1_cd, True)
    z_cell1 = _fused_linear(agg_c1, x_cell, W_rel1_dc, W_root1_dc, b1_dc, True)
    z_drug1, z_cell1 = lax.optimization_barrier((z_drug1, z_cell1))

    agg_d2 = _seg_sum3(z_cell1, src_cd, dst_cd, edge_weight_c2d, NP)
    agg_c2 = _seg_sum3(z_drug1, src_dc, dst_dc, edge_weight_d2c, NP)
    agg_d2, agg_c2 = lax.optimization_barrier((agg_d2, agg_c2))
    z_drug2 = _fused_linear(agg_d2, z_drug1, W_rel2_cd, W_root2_cd, b2_cd, False)
    z_cell2 = _fused_linear(agg_c2, z_cell1, W_rel2_dc, W_root2_dc, b2_dc, False)
    z_drug2, z_cell2 = lax.optimization_barrier((z_drug2, z_cell2))

    row1 = edge_label_index[0]
    row2 = edge_label_index[1]
    return (jnp.take(z_cell2, row1, axis=0) * jnp.take(z_drug2, row2, axis=0)).sum(axis=-1)


# speedup vs baseline: 2.0295x; 1.1928x over previous
import functools

import jax
import jax.numpy as jnp
from jax import lax
from jax.experimental import pallas as pl
from jax.experimental.pallas import tpu as pltpu
from jax.experimental.pallas import tpu_sc as plsc

N_CELL = 10000
N_DRUG = 10000
E = 160000
D_IN = 256
H = 512

NC = 2
NS = 16
LANES = 16
NP = 10240
EPAD = 163840


def _mm_body(relu, ng, a_ref, b_ref, w1_ref, w2_ref, bias_ref, o_ref):
    acc = jnp.dot(a_ref[0], w1_ref[0], preferred_element_type=jnp.float32)
    for g in range(1, ng):
        acc += jnp.dot(a_ref[g], w1_ref[g], preferred_element_type=jnp.float32)
    acc += jnp.dot(b_ref[...], w2_ref[...], preferred_element_type=jnp.float32)
    acc += bias_ref[...]
    o_ref[...] = jnp.maximum(acc, 0.0) if relu else acc


def _fused_linear(agg3, x, w_rel, w_root, bias, relu):
    ng, n, _ = agg3.shape
    k = ng * 128
    h = w_rel.shape[1]
    w3 = w_rel.reshape(ng, 128, h)
    blk = 1024
    grid = (n // blk,)
    return pl.pallas_call(
        functools.partial(_mm_body, relu, ng),
        grid=grid,
        in_specs=[
            pl.BlockSpec((ng, blk, 128), lambda i: (0, i, 0)),
            pl.BlockSpec((blk, k), lambda i: (i, 0)),
            pl.BlockSpec((ng, 128, h), lambda i: (0, 0, 0)),
            pl.BlockSpec((k, h), lambda i: (0, 0)),
            pl.BlockSpec((1, h), lambda i: (0, 0)),
        ],
        out_specs=pl.BlockSpec((blk, h), lambda i: (i, 0)),
        out_shape=jax.ShapeDtypeStruct((n, h), jnp.float32),
    )(agg3, x, w3, w_root, bias.reshape(1, h))


def _sc_segsum_call(num_dst, G):
    P = G // NC
    EP = EPAD // NS
    K = 128
    NCH = EP // K
    RPT = num_dst // NS
    mesh = plsc.VectorSubcoreMesh(core_axis_name="c", subcore_axis_name="s")

    buf_types = []
    for _ in range(2):
        buf_types += [
            pltpu.VMEM((K,), jnp.int32),
            pltpu.VMEM((K,), jnp.int32),
            pltpu.VMEM((K,), jnp.int32),
            pltpu.VMEM((K,), jnp.float32),
            pltpu.VMEM((K, 128), jnp.float32),
            pltpu.SemaphoreType.DMA,
            pltpu.SemaphoreType.DMA,
        ]

    @functools.partial(
        pl.kernel,
        out_type=jax.ShapeDtypeStruct((G, num_dst, 128), jnp.float32),
        mesh=mesh,
        scratch_types=buf_types + [
            pltpu.VMEM_SHARED((num_dst, 128), jnp.float32),
        ],
    )
    def k(xv_h, src_h, dst_h, w_h, out_h, *scr):
        bufs = (scr[0:7], scr[7:14])
        acc = scr[14]
        c = lax.axis_index("c")
        s = lax.axis_index("s")
        tbase = s * EP
        rbase = s * RPT

        def issue_idx(ch, b):
            base = tbase + ch * K
            pltpu.async_copy(src_h.at[pl.ds(base, K)], b[0], b[5])
            pltpu.async_copy(dst_h.at[pl.ds(base, K)], b[1], b[5])
            pltpu.async_copy(w_h.at[pl.ds(base, K)], b[3], b[5])

        def wait_idx(ch, b):
            base = tbase + ch * K
            pltpu.make_async_copy(src_h.at[pl.ds(base, K)], b[0], b[5]).wait()
            pltpu.make_async_copy(dst_h.at[pl.ds(base, K)], b[1], b[5]).wait()
            pltpu.make_async_copy(w_h.at[pl.ds(base, K)], b[3], b[5]).wait()

        def issue_gather(g, b):
            def gix(i, _):
                v = b[0][pl.ds(i * 16, 16)]
                b[2][pl.ds(i * 16, 16)] = v * G + g
                return _
            lax.fori_loop(0, K // 16, gix, None)
            pltpu.async_copy(xv_h.at[b[2]], b[4], b[6])

        def wait_gather(b):
            pltpu.make_async_copy(xv_h.at[b[2]], b[4], b[6]).wait()

        def scale_scatter(b):
            rows, wbuf = b[4], b[3]

            def scale(i, _):
                w16 = wbuf[pl.ds(i * 16, 16)]
                for l in range(16):
                    wl = w16[l]
                    e = i * 16 + l
                    for j in range(8):
                        rows[e, pl.ds(16 * j, 16)] = (
                            rows[e, pl.ds(16 * j, 16)] * wl)
                return _
            lax.fori_loop(0, K // 16, scale, None)
            pltpu.sync_copy(rows, acc.at[b[1]], add=True)

        for p in range(P):
            g = p * NC + c
            rows0 = bufs[0][4]

            def zb(i, _):
                for j in range(8):
                    rows0[i, pl.ds(16 * j, 16)] = jnp.zeros((16,), jnp.float32)
                return _
            lax.fori_loop(0, K, zb, None)
            for q in range(RPT // K):
                pltpu.sync_copy(rows0, acc.at[pl.ds(rbase + q * K, K)])
            plsc.subcore_barrier()

            issue_idx(0, bufs[0])
            wait_idx(0, bufs[0])
            issue_gather(g, bufs[0])
            issue_idx(1, bufs[1])

            def body(ch, b, bn):
                @pl.when(ch + 1 < NCH)
                def _():
                    wait_idx(ch + 1, bn)
                    issue_gather(g, bn)
                wait_gather(b)
                scale_scatter(b)

                @pl.when(ch + 2 < NCH)
                def _():
                    issue_idx(ch + 2, b)

            def pair(ch2, _):
                body(2 * ch2, bufs[0], bufs[1])
                body(2 * ch2 + 1, bufs[1], bufs[0])
                return _
            lax.fori_loop(0, NCH // 2, pair, None)

            plsc.subcore_barrier()
            pltpu.sync_copy(acc.at[pl.ds(rbase, RPT)],
                            out_h.at[g, pl.ds(rbase, RPT)])
            if p + 1 < P:
                plsc.subcore_barrier()

    return k


def _seg_sum3(x_src, src, dst, w, num_dst):
    n_src, d = x_src.shape
    G = d // 128
    xv = x_src.reshape(n_src * G, 128)
    pad = EPAD - src.shape[0]
    if pad:
        src = jnp.pad(src, (0, pad))
        dst = jnp.pad(dst, (0, pad))
        w = jnp.pad(w, (0, pad))
    return _sc_segsum_call(num_dst, G)(xv, src, dst, w)


def kernel(x_cell, x_drug, edge_index_c2d, edge_weight_c2d, edge_index_d2c,
           edge_weight_d2c, edge_label_index, W_rel1_cd, W_root1_cd, b1_cd,
           W_rel1_dc, W_root1_dc, b1_dc, W_rel2_cd, W_root2_cd, b2_cd,
           W_rel2_dc, W_root2_dc, b2_dc):
    src_cd, dst_cd = edge_index_c2d[0], edge_index_c2d[1]
    src_dc, dst_dc = edge_index_d2c[0], edge_index_d2c[1]

    x_cell = jnp.pad(x_cell, ((0, NP - N_CELL), (0, 0)))
    x_drug = jnp.pad(x_drug, ((0, NP - N_DRUG), (0, 0)))

    agg_d1 = _seg_sum3(x_cell, src_cd, dst_cd, edge_weight_c2d, NP)
    agg_c1 = _seg_sum3(x_drug, src_dc, dst_dc, edge_weight_d2c, NP)
    agg_d1, agg_c1 = lax.optimization_barrier((agg_d1, agg_c1))
    z_drug1 = _fused_linear(agg_d1, x_drug, W_rel1_cd, W_root1_cd, b1_cd, True)
    z_cell1 = _fused_linear(agg_c1, x_cell, W_rel1_dc, W_root1_dc, b1_dc, True)
    z_drug1, z_cell1 = lax.optimization_barrier((z_drug1, z_cell1))

    agg_d2 = _seg_sum3(z_cell1, src_cd, dst_cd, edge_weight_c2d, NP)
    agg_c2 = _seg_sum3(z_drug1, src_dc, dst_dc, edge_weight_d2c, NP)
    agg_d2, agg_c2 = lax.optimization_barrier((agg_d2, agg_c2))
    z_drug2 = _fused_linear(agg_d2, z_drug1, W_rel2_cd, W_root2_cd, b2_cd, False)
    z_cell2 = _fused_linear(agg_c2, z_cell1, W_rel2_dc, W_root2_dc, b2_dc, False)
    z_drug2, z_cell2 = lax.optimization_barrier((z_drug2, z_cell2))

    row1 = edge_label_index[0]
    row2 = edge_label_index[1]
    return (jnp.take(z_cell2, row1, axis=0) * jnp.take(z_drug2, row2, axis=0)).sum(axis=-1)
